# baseline (speedup 1.0000x reference)
import functools

import jax
import jax.numpy as jnp
from jax import lax
from jax.experimental import pallas as pl
from jax.experimental.pallas import tpu as pltpu
from jax.experimental.pallas import tpu_sc as plsc

B, L, D = 4096, 200, 64
N = B * L
NC, NS = 2, 16
NW = NC * NS
PER_W = N // NW
CHUNK = 128
NCHUNK = PER_W // CHUNK

_mesh = plsc.VectorSubcoreMesh(core_axis_name="c", subcore_axis_name="s")


@functools.partial(
    pl.kernel,
    out_type=jax.ShapeDtypeStruct((N, D), jnp.float32),
    mesh=_mesh,
    compiler_params=pltpu.CompilerParams(use_tc_tiling_on_sc=False),
    scratch_types=[
        pltpu.VMEM((NCHUNK, CHUNK), jnp.int32),
        pltpu.VMEM((CHUNK, D), jnp.float32),
        pltpu.SemaphoreType.DMA,
    ],
)
def _gather(idx_hbm, table_hbm, out_hbm, idx_v, rows_v, sem):
  wid = lax.axis_index("s") * NC + lax.axis_index("c")
  base = wid * PER_W
  pltpu.sync_copy(idx_hbm.at[wid], idx_v)

  def body(g, carry):
    pltpu.async_copy(table_hbm.at[idx_v.at[g]], rows_v, sem).wait()
    pltpu.sync_copy(rows_v, out_hbm.at[pl.ds(base + g * CHUNK, CHUNK)])
    return carry

  lax.fori_loop(0, NCHUNK, body, 0)


def kernel(indices, table):
  idx = indices.reshape(NW, NCHUNK, CHUNK).astype(jnp.int32)
  out = _gather(idx, table)
  return out.reshape(B, L, D)


# speedup vs baseline: 1.1138x; 1.1138x over previous
import functools

import jax
import jax.numpy as jnp
from jax import lax
from jax.experimental import pallas as pl
from jax.experimental.pallas import tpu as pltpu
from jax.experimental.pallas import tpu_sc as plsc

B, L, D = 4096, 200, 64
N = B * L
NC, NS = 2, 16
NW = NC * NS
PER_W = N // NW
CHUNK = 128
NCHUNK = PER_W // CHUNK
K = 4
GROUP = K * CHUNK
NT = NCHUNK // K

_mesh = plsc.VectorSubcoreMesh(core_axis_name="c", subcore_axis_name="s")


@functools.partial(
    pl.kernel,
    out_type=jax.ShapeDtypeStruct((N, D), jnp.float32),
    mesh=_mesh,
    compiler_params=pltpu.CompilerParams(use_tc_tiling_on_sc=False),
    scratch_types=[
        pltpu.VMEM((NCHUNK, CHUNK), jnp.int32),
        pltpu.VMEM((2, GROUP, D), jnp.float32),
        pltpu.SemaphoreType.DMA,
        pltpu.SemaphoreType.DMA,
    ],
)
def _gather(idx_hbm, table_hbm, out_hbm, idx_v, rows_v, gsem, ssem):
  wid = lax.axis_index("s") * NC + lax.axis_index("c")
  base = wid * PER_W
  pltpu.sync_copy(idx_hbm.at[wid], idx_v)

  def start_group(t, p):
    for j in range(K):
      pltpu.async_copy(
          table_hbm.at[idx_v.at[t * K + j]],
          rows_v.at[p, pl.ds(j * CHUNK, CHUNK)],
          gsem,
      )

  def wait_gathers():
    pltpu.make_async_copy(out_hbm.at[pl.ds(0, GROUP)], rows_v.at[0], gsem).wait()

  def wait_store():
    pltpu.make_async_copy(rows_v.at[0], out_hbm.at[pl.ds(0, GROUP)], ssem).wait()

  start_group(0, 0)

  def body(t, carry):
    p = lax.rem(t, 2)
    wait_gathers()

    @pl.when(t + 1 < NT)
    def _():
      @pl.when(t >= 1)
      def _():
        wait_store()
      start_group(t + 1, 1 - p)

    pltpu.async_copy(
        rows_v.at[p],
        out_hbm.at[pl.ds(base + t * GROUP, GROUP)],
        ssem,
    )
    return carry

  lax.fori_loop(0, NT, body, 0)
  wait_store()
  wait_store()


def kernel(indices, table):
  idx = indices.reshape(NW, NCHUNK, CHUNK).astype(jnp.int32)
  out = _gather(idx, table)
  return out.reshape(B, L, D)
